# Initial kernel scaffold; baseline (speedup 1.0000x reference)
import jax
import jax.numpy as jnp
from jax.experimental import pallas as pl


def kernel(protos):
    raise NotImplementedError("write your pallas kernel here")


# speedup vs baseline: 2.1782x; 2.1782x over previous
import jax
import jax.numpy as jnp
from jax import lax
from jax.experimental import pallas as pl
from jax.experimental.pallas import tpu as pltpu
from jax.experimental.pallas import tpu_sc as plsc

_N, _C, _H, _W = 16, 96, 224, 224
_RC, _CC = 4, 4
_HC, _WC = _H // _RC, _W // _CC

_NCORES = 2
_NSUB = 16
_NW = _NCORES * _NSUB
_CG = 4
_PAIRS_PER_W = (_N * _RC) // _NW


def _permutation_table():
    key = jax.random.key(42)
    keys = jax.random.split(key, _RC * _CC)
    return jnp.stack(
        [jax.random.permutation(keys[i], _N) for i in range(_RC * _CC)], 0)


def _shuffle_body(protos, srcs, out, srcs_v, buf):
    wid = lax.axis_index("s") * _NCORES + lax.axis_index("c")
    pltpu.sync_copy(srcs, srcs_v)

    for p in range(_PAIRS_PER_W):
        pair = wid * _PAIRS_PER_W + p
        n = pair // _RC
        r = pair % _RC
        src_row = srcs_v[pair]

        def body(cg, _):
            cbase = cg * _CG
            for wc in range(_CC):
                src = src_row[wc]
                pltpu.sync_copy(
                    protos.at[src, pl.ds(cbase, _CG),
                              pl.ds(r * _HC, _HC), pl.ds(wc * _WC, _WC)],
                    buf.at[:, :, pl.ds(wc * _WC, _WC)],
                )
            pltpu.sync_copy(
                buf,
                out.at[n, pl.ds(cbase, _CG), pl.ds(r * _HC, _HC), :])
            return 0

        lax.fori_loop(0, _C // _CG, body, 0)


def _shuffle(protos, srcs):
    mesh = plsc.VectorSubcoreMesh(core_axis_name="c", subcore_axis_name="s")
    f = pl.kernel(
        _shuffle_body,
        out_type=jax.ShapeDtypeStruct((_N, _C, _H, _W), jnp.float32),
        mesh=mesh,
        scratch_types=[
            pltpu.VMEM((_N * _RC, 16), jnp.int32),
            pltpu.VMEM((_CG, _HC, _W), jnp.float32),
        ],
        compiler_params=pltpu.CompilerParams(use_tc_tiling_on_sc=False),
    )
    return f(protos, srcs)


def kernel(protos):
    protos = lax.stop_gradient(protos)
    idxs = _permutation_table()
    srcs = jnp.transpose(idxs.reshape(_RC, _CC, _N), (2, 0, 1)).reshape(
        _N * _RC, _CC)
    srcs = jnp.pad(srcs, ((0, 0), (0, 16 - _CC)))
    spro = _shuffle(protos, srcs)
    return spro, idxs
